# baseline (speedup 1.0000x reference)
import jax
import jax.numpy as jnp
from jax.experimental import pallas as pl
from jax.experimental.pallas import tpu as pltpu

B, H, N, F_IN, F_OUT = 8, 12, 1024, 192, 192
DH = F_OUT // H
BN = 256
NI = N // BN
TOT = B * NI


def _gcn_body(x_ref, adj_hbm, w_ref, b_ref, h_ref, adjc_hbm,
              support_ref, abuf, sem_in, sem_out):
    bi = pl.program_id(0)
    i = pl.program_id(1)
    k = bi * NI + i

    def rd(kk):
        return pltpu.make_async_copy(
            adj_hbm.at[kk // NI, :, pl.ds((kk % NI) * BN, BN), :],
            abuf.at[kk % NI],
            sem_in.at[kk % NI],
        )

    def wr(kk):
        return pltpu.make_async_copy(
            abuf.at[kk % NI],
            adjc_hbm.at[kk // NI, :, pl.ds((kk % NI) * BN, BN), :],
            sem_out.at[kk % NI],
        )

    @pl.when(k == 0)
    def _():
        rd(0).start()
        rd(1).start()
        rd(2).start()

    @pl.when(i == 0)
    def _():
        support_ref[...] = jnp.dot(
            x_ref[0].astype(jnp.bfloat16),
            w_ref[...].astype(jnp.bfloat16),
            preferred_element_type=jnp.float32,
        )

    rd(k).wait()
    wr(k).start()

    parts = []
    for h in range(H):
        a = abuf[i, h].astype(jnp.bfloat16)
        s = support_ref[:, h * DH:(h + 1) * DH]
        parts.append(
            jnp.dot(a, s.astype(jnp.bfloat16),
                    preferred_element_type=jnp.float32)
        )
    acc = jnp.concatenate(parts, axis=-1)
    h_ref[0] = jax.nn.gelu(acc + b_ref[...])

    @pl.when(k >= 1)
    def _():
        wr(k - 1).wait()

    @pl.when(k + 3 < TOT)
    def _():
        rd(k + 3).start()

    @pl.when(k == TOT - 1)
    def _():
        wr(TOT - 1).wait()


@jax.jit
def kernel(x, adj, W, b):
    b2 = b.reshape(1, F_OUT)
    grid = (B, NI)
    h_out, adjc = pl.pallas_call(
        _gcn_body,
        grid=grid,
        in_specs=[
            pl.BlockSpec((1, N, F_IN), lambda bi, i: (bi, 0, 0)),
            pl.BlockSpec(memory_space=pl.ANY),
            pl.BlockSpec((F_IN, F_OUT), lambda bi, i: (0, 0)),
            pl.BlockSpec((1, F_OUT), lambda bi, i: (0, 0)),
        ],
        out_specs=[
            pl.BlockSpec((1, BN, F_OUT), lambda bi, i: (bi, i, 0)),
            pl.BlockSpec(memory_space=pl.ANY),
        ],
        out_shape=[
            jax.ShapeDtypeStruct((B, N, F_OUT), jnp.float32),
            jax.ShapeDtypeStruct((B, H, N, N), jnp.float32),
        ],
        scratch_shapes=[
            pltpu.VMEM((N, F_OUT), jnp.float32),
            pltpu.VMEM((NI, H, BN, N), jnp.float32),
            pltpu.SemaphoreType.DMA((NI,)),
            pltpu.SemaphoreType.DMA((NI,)),
        ],
    )(x, adj, W, b2)
    return h_out, adjc.reshape(B * H, N, N)
